# baseline (speedup 1.0000x reference)
import functools

import jax
import jax.numpy as jnp
from jax import lax
from jax.experimental import pallas as pl
from jax.experimental.pallas import tpu as pltpu
from jax.experimental.pallas import tpu_sc as plsc

NUM_EMB = 100001
EMBED_DIM = 32
BATCH = 16384
NUM_CORES = 2
NUM_SUBCORES = 16
NUM_WORKERS = NUM_CORES * NUM_SUBCORES
BPW = BATCH // NUM_WORKERS
CHUNK = 128
CPR = BPW // CHUNK
NCHUNK = EMBED_DIM * CPR
GROUP = 12

_mesh = plsc.VectorSubcoreMesh(core_axis_name="c", subcore_axis_name="s")


@functools.partial(
    pl.kernel,
    out_type=jax.ShapeDtypeStruct((EMBED_DIM, BATCH), jnp.float32),
    mesh=_mesh,
    scratch_types=[
        pltpu.VMEM((BPW,), jnp.int32),
        pltpu.VMEM((EMBED_DIM, BPW), jnp.int32),
        pltpu.VMEM((EMBED_DIM, BPW), jnp.float32),
        pltpu.SemaphoreType.DMA,
    ],
    compiler_params=pltpu.CompilerParams(use_tc_tiling_on_sc=False),
)
def _sc_gather_t(idx_hbm, t1d_hbm, outT_hbm, idx_v, widx_v, outv, sem):
    wid = lax.axis_index("s") * NUM_CORES + lax.axis_index("c")
    base = wid * BPW

    pltpu.sync_copy(idx_hbm.at[pl.ds(base, BPW)], idx_v)

    copies = []

    def start(c):
        j, cc = c // CPR, c % CPR
        copies.append(
            pltpu.async_copy(
                t1d_hbm.at[widx_v.at[j, pl.ds(cc * CHUNK, CHUNK)]],
                outv.at[j, pl.ds(cc * CHUNK, CHUNK)],
                sem,
            )
        )

    for j in range(EMBED_DIM):
        for u in range(BPW // 16):
            widx_v[j, pl.ds(u * 16, 16)] = idx_v[pl.ds(u * 16, 16)] + j * NUM_EMB
        for cc in range(CPR):
            c = j * CPR + cc
            start(c)
            if c >= GROUP:
                copies[c - GROUP].wait()
    for c in range(NCHUNK - GROUP, NCHUNK):
        copies[c].wait()

    pltpu.sync_copy(outv, outT_hbm.at[:, pl.ds(base, BPW)])


def kernel(titles, table):
    t1d = table.T.reshape(-1)
    outT = _sc_gather_t(titles.astype(jnp.int32), t1d)
    return outT.T


# speedup vs baseline: 1.0653x; 1.0653x over previous
import functools

import jax
import jax.numpy as jnp
from jax import lax
from jax.experimental import pallas as pl
from jax.experimental.pallas import tpu as pltpu
from jax.experimental.pallas import tpu_sc as plsc

NUM_EMB = 100001
EMBED_DIM = 32
BATCH = 16384
NUM_CORES = 2
NUM_SUBCORES = 16
NUM_WORKERS = NUM_CORES * NUM_SUBCORES
BPW = BATCH // NUM_WORKERS
CHUNK = 128
CPR = BPW // CHUNK
NCHUNK = EMBED_DIM * CPR
GROUP = 32

_mesh = plsc.VectorSubcoreMesh(core_axis_name="c", subcore_axis_name="s")


@functools.partial(
    pl.kernel,
    out_type=jax.ShapeDtypeStruct((EMBED_DIM, BATCH), jnp.float32),
    mesh=_mesh,
    scratch_types=[
        pltpu.VMEM((BPW,), jnp.int32),
        pltpu.VMEM((EMBED_DIM, BPW), jnp.int32),
        pltpu.VMEM((EMBED_DIM, BPW), jnp.float32),
        pltpu.SemaphoreType.DMA,
    ],
    compiler_params=pltpu.CompilerParams(use_tc_tiling_on_sc=False),
)
def _sc_gather_t(idx_hbm, t1d_hbm, outT_hbm, idx_v, widx_v, outv, sem):
    wid = lax.axis_index("s") * NUM_CORES + lax.axis_index("c")
    base = wid * BPW

    pltpu.sync_copy(idx_hbm.at[pl.ds(base, BPW)], idx_v)

    copies = []

    def start(c):
        j, cc = c // CPR, c % CPR
        copies.append(
            pltpu.async_copy(
                t1d_hbm.at[widx_v.at[j, pl.ds(cc * CHUNK, CHUNK)]],
                outv.at[j, pl.ds(cc * CHUNK, CHUNK)],
                sem,
            )
        )

    def build(u, carry):
        v = idx_v[pl.ds(u * 16, 16)]
        for j in range(EMBED_DIM):
            widx_v[j, pl.ds(u * 16, 16)] = v + j * NUM_EMB
        return carry

    lax.fori_loop(0, BPW // 16, build, 0)

    for c in range(NCHUNK):
        start(c)
        if c >= GROUP:
            copies[c - GROUP].wait()
    for c in range(NCHUNK - GROUP, NCHUNK):
        copies[c].wait()

    pltpu.sync_copy(outv, outT_hbm.at[:, pl.ds(base, BPW)])


def kernel(titles, table):
    t1d = table.T.reshape(-1)
    outT = _sc_gather_t(titles.astype(jnp.int32), t1d)
    return outT.T


# speedup vs baseline: 1.1226x; 1.0538x over previous
import functools

import jax
import jax.numpy as jnp
from jax import lax
from jax.experimental import pallas as pl
from jax.experimental.pallas import tpu as pltpu
from jax.experimental.pallas import tpu_sc as plsc

NUM_EMB = 100001
EMBED_DIM = 32
BATCH = 16384
NUM_CORES = 2
NUM_SUBCORES = 16
NUM_WORKERS = NUM_CORES * NUM_SUBCORES
BPW = BATCH // NUM_WORKERS
CHUNK = 512
CPR = BPW // CHUNK
NCHUNK = EMBED_DIM * CPR
GROUP = 16

_mesh = plsc.VectorSubcoreMesh(core_axis_name="c", subcore_axis_name="s")


@functools.partial(
    pl.kernel,
    out_type=jax.ShapeDtypeStruct((EMBED_DIM, BATCH), jnp.float32),
    mesh=_mesh,
    scratch_types=[
        pltpu.VMEM((BPW,), jnp.int32),
        pltpu.VMEM((EMBED_DIM, BPW), jnp.int32),
        pltpu.VMEM((EMBED_DIM, BPW), jnp.float32),
        pltpu.SemaphoreType.DMA,
    ],
    compiler_params=pltpu.CompilerParams(use_tc_tiling_on_sc=False),
)
def _sc_gather_t(idx_hbm, t1d_hbm, outT_hbm, idx_v, widx_v, outv, sem):
    wid = lax.axis_index("s") * NUM_CORES + lax.axis_index("c")
    base = wid * BPW

    pltpu.sync_copy(idx_hbm.at[pl.ds(base, BPW)], idx_v)

    copies = []

    def start(c):
        j, cc = c // CPR, c % CPR
        copies.append(
            pltpu.async_copy(
                t1d_hbm.at[widx_v.at[j, pl.ds(cc * CHUNK, CHUNK)]],
                outv.at[j, pl.ds(cc * CHUNK, CHUNK)],
                sem,
            )
        )

    def build(u, carry):
        v = idx_v[pl.ds(u * 16, 16)]
        for j in range(EMBED_DIM):
            widx_v[j, pl.ds(u * 16, 16)] = v + j * NUM_EMB
        return carry

    lax.fori_loop(0, BPW // 16, build, 0)

    for c in range(NCHUNK):
        start(c)
        if c >= GROUP:
            copies[c - GROUP].wait()
    for c in range(NCHUNK - GROUP, NCHUNK):
        copies[c].wait()

    pltpu.sync_copy(outv, outT_hbm.at[:, pl.ds(base, BPW)])


def kernel(titles, table):
    t1d = table.T.reshape(-1)
    outT = _sc_gather_t(titles.astype(jnp.int32), t1d)
    return outT.T
